# Initial kernel scaffold; baseline (speedup 1.0000x reference)
import jax
import jax.numpy as jnp
from jax.experimental import pallas as pl


def kernel(predictions, anchors):
    raise NotImplementedError("write your pallas kernel here")


# speedup vs baseline: 16.8219x; 16.8219x over previous
import functools

import jax
import jax.numpy as jnp
from jax import lax
from jax.experimental import pallas as pl
from jax.experimental.pallas import tpu as pltpu
from jax.experimental.pallas import tpu_sc as plsc

B = 8
N = 20000
C = 80
ROWS = B * C
K = 500
KPAD = 512
BUF = 544
MAXDET = 100
IOU_THR = 0.6
SCORE_THR = 0.1
NEG = -1e30


def _thresh_body(logit_ref, t_ref, bud_ref):
    x = logit_ref[...]
    bu = lax.bitcast_convert_type(x, jnp.uint32)
    topbit = jnp.uint32(0x80000000)
    ukey = jnp.where(bu >= topbit, ~bu, bu | topbit)

    def body(i, prefix):
        bit = jnp.uint32(31) - lax.convert_element_type(i, jnp.uint32)
        trial = prefix | lax.shift_left(jnp.uint32(1), bit)
        cnt = jnp.sum((ukey >= trial).astype(jnp.int32), axis=1, keepdims=True)
        return jnp.where(cnt >= K, trial, prefix)

    prefix = lax.fori_loop(0, 32, body, jnp.zeros((x.shape[0], 1), jnp.uint32))
    cnt_gt = jnp.sum((ukey > prefix).astype(jnp.int32), axis=1, keepdims=True)
    bits = jnp.where(prefix >= topbit, prefix ^ topbit, ~prefix)
    t = lax.bitcast_convert_type(bits, jnp.float32)
    t_ref[...] = jnp.broadcast_to(t, t_ref.shape)
    bud_ref[...] = jnp.broadcast_to(K - cnt_gt, bud_ref.shape)


def _thresh_call(logits_t):
    R = 64
    return pl.pallas_call(
        _thresh_body,
        grid=(ROWS // R,),
        in_specs=[pl.BlockSpec((R, N), lambda i: (i, 0))],
        out_specs=[
            pl.BlockSpec((R, 16), lambda i: (i, 0)),
            pl.BlockSpec((R, 16), lambda i: (i, 0)),
        ],
        out_shape=[
            jax.ShapeDtypeStruct((ROWS, 16), jnp.float32),
            jax.ShapeDtypeStruct((ROWS, 16), jnp.int32),
        ],
    )(logits_t)


def _decode_body(cx_ref, cy_ref, w_ref, h_ref, acx_ref, acy_ref, aw_ref, ah_ref,
                 x1_ref, y1_ref, x2_ref, y2_ref):
    v01 = jnp.float32(0.1)
    v2 = jnp.float32(0.2)
    aw = aw_ref[...]
    ah = ah_ref[...]
    xx = (cx_ref[...] * v01) * aw + acx_ref[...]
    yy = (cy_ref[...] * v01) * ah + acy_ref[...]
    ww = jnp.exp(w_ref[...] * v2) * aw
    hh = jnp.exp(h_ref[...] * v2) * ah
    x1_ref[...] = xx - ww / 2.0
    y1_ref[...] = yy - hh / 2.0
    x2_ref[...] = xx + ww / 2.0
    y2_ref[...] = yy + hh / 2.0


def _decode_call(cx, cy, w, h, acx, acy, aw, ah):
    spec_b = pl.BlockSpec((B, N), lambda: (0, 0))
    spec_a = pl.BlockSpec((1, N), lambda: (0, 0))
    return pl.pallas_call(
        _decode_body,
        in_specs=[spec_b] * 4 + [spec_a] * 4,
        out_specs=[spec_b] * 4,
        out_shape=[jax.ShapeDtypeStruct((B, N), jnp.float32)] * 4,
    )(cx, cy, w, h, acx, acy, aw, ah)


def _sc_body(logits_hbm, t_hbm, bud_hbm, x1_hbm, y1_hbm, x2_hbm, y2_hbm,
             clog_hbm, cx1_hbm, cy1_hbm, cx2_hbm, cy2_hbm,
             row_v, p1_v, p2_v, p3_v, p4_v, tb_v, bb_v,
             logit_b, idx_b, eq_b, o1_b, o2_b, o3_b, o4_b):
    wid = lax.axis_index("s") * 2 + lax.axis_index("c")
    batch = wid // 4
    row0 = batch * C + (wid % 4) * 20

    pltpu.sync_copy(x1_hbm.at[batch], p1_v)
    pltpu.sync_copy(y1_hbm.at[batch], p2_v)
    pltpu.sync_copy(x2_hbm.at[batch], p3_v)
    pltpu.sync_copy(y2_hbm.at[batch], p4_v)
    toff = row0 % 8
    tbase = pl.multiple_of(row0 - toff, 8)
    pltpu.sync_copy(t_hbm.at[pl.ds(tbase, 24), :], tb_v)
    pltpu.sync_copy(bud_hbm.at[pl.ds(tbase, 24), :], bb_v)

    iota = lax.iota(jnp.int32, 16)
    negv = jnp.full((16,), NEG, jnp.float32)
    zidx = jnp.zeros((16,), jnp.int32)

    def do_row(r, _):
        row = row0 + r
        pltpu.sync_copy(logits_hbm.at[row], row_v)
        t_vec = tb_v[toff + r, :]
        bud_vec = bb_v[toff + r, :]

        def init(j, _):
            logit_b[pl.ds(j * 16, 16)] = negv
            idx_b[pl.ds(j * 16, 16)] = zidx
            return 0

        lax.fori_loop(0, BUF // 16, init, 0)

        def step(i, carry):
            cur, ec = carry
            v = row_v[pl.ds(i * 16, 16)]
            m_gt = v > t_vec
            m_eq = v == t_vec
            any_sel = plsc.all_reduce_population_count(m_gt | m_eq)[0] > 0

            def taken(c):
                cur0, ec0 = c
                base = iota + i * 16
                plsc.store_compressed(idx_b.at[pl.ds(cur0, 16)], base, mask=m_gt)
                plsc.store_compressed(logit_b.at[pl.ds(cur0, 16)], v, mask=m_gt)
                ngt = plsc.all_reduce_population_count(m_gt)[0]

                def take_eq(e):
                    plsc.store_compressed(eq_b.at[pl.ds(e, 16)], base, mask=m_eq)
                    return e + plsc.all_reduce_population_count(m_eq)[0]

                ec1 = lax.cond(ec0 < KPAD, take_eq, lambda e: e, ec0)
                return (cur0 + ngt, ec1)

            return lax.cond(any_sel, taken, lambda c: c, (cur, ec))

        cur, _ec = lax.fori_loop(0, N // 16, step, (0, 0))

        budget = bud_vec[0]

        def fill(j, _):
            base = j * 16
            m = (iota + base) < bud_vec
            vi = eq_b[pl.ds(base, 16)]
            plsc.store_compressed(idx_b.at[pl.ds(cur + base, 16)], vi, mask=m)
            plsc.store_compressed(logit_b.at[pl.ds(cur + base, 16)], t_vec, mask=m)
            return 0

        lax.fori_loop(0, (budget + 15) // 16, fill, 0)

        def gather(j, _):
            s = pl.ds(j * 16, 16)
            vi = idx_b[s]
            o1_b[s] = plsc.load_gather(p1_v, [vi])
            o2_b[s] = plsc.load_gather(p2_v, [vi])
            o3_b[s] = plsc.load_gather(p3_v, [vi])
            o4_b[s] = plsc.load_gather(p4_v, [vi])
            return 0

        lax.fori_loop(0, KPAD // 16, gather, 0)

        pltpu.sync_copy(logit_b.at[pl.ds(0, KPAD)], clog_hbm.at[row])
        pltpu.sync_copy(o1_b, cx1_hbm.at[row])
        pltpu.sync_copy(o2_b, cy1_hbm.at[row])
        pltpu.sync_copy(o3_b, cx2_hbm.at[row])
        pltpu.sync_copy(o4_b, cy2_hbm.at[row])
        return 0

    lax.fori_loop(0, 20, do_row, 0)


def _sc_compact(logits_t, tb, bb, x1p, y1p, x2p, y2p):
    f32 = jnp.float32
    mesh = plsc.VectorSubcoreMesh(core_axis_name="c", subcore_axis_name="s")
    return pl.kernel(
        _sc_body,
        out_type=[jax.ShapeDtypeStruct((ROWS, KPAD), f32)] * 5,
        mesh=mesh,
        compiler_params=pltpu.CompilerParams(needs_layout_passes=False),
        scratch_types=[
            pltpu.VMEM((N,), f32),
            pltpu.VMEM((N,), f32),
            pltpu.VMEM((N,), f32),
            pltpu.VMEM((N,), f32),
            pltpu.VMEM((N,), f32),
            pltpu.VMEM((24, 16), f32),
            pltpu.VMEM((24, 16), jnp.int32),
            pltpu.VMEM((BUF,), f32),
            pltpu.VMEM((BUF,), jnp.int32),
            pltpu.VMEM((BUF,), jnp.int32),
            pltpu.VMEM((KPAD,), f32),
            pltpu.VMEM((KPAD,), f32),
            pltpu.VMEM((KPAD,), f32),
            pltpu.VMEM((KPAD,), f32),
        ],
    )(logits_t, tb, bb, x1p, y1p, x2p, y2p)


def _nms_body(clog_ref, cx1_ref, cy1_ref, cx2_ref, cy2_ref,
              fs_ref, fx1_ref, fy1_ref, fx2_ref, fy2_ref, fc_ref, fv_ref):
    logit = clog_ref[...]
    scores = 1.0 / (1.0 + jnp.exp(-logit))
    x1 = cx1_ref[...]
    y1 = cy1_ref[...]
    x2 = cx2_ref[...]
    y2 = cy2_ref[...]
    a2 = jnp.maximum(x2 - x1, 0.0) * jnp.maximum(y2 - y1, 0.0)
    lane = lax.broadcasted_iota(jnp.int32, (C, KPAD), 1)
    thr = jnp.float32(SCORE_THR)

    sstep = lax.broadcasted_iota(jnp.int32, (C, MAXDET), 1)

    def nms_step(s, carry):
        af, os_, ox1, oy1, ox2, oy2 = carry
        masked = jnp.where(af > 0.0, scores, -1.0)
        mx = jnp.max(masked, axis=1, keepdims=True)
        am = jnp.min(jnp.where(masked == mx, lane, N), axis=1, keepdims=True)
        onehot = lane == am
        valid = mx > thr
        sx1 = jnp.sum(jnp.where(onehot, x1, 0.0), axis=1, keepdims=True)
        sy1 = jnp.sum(jnp.where(onehot, y1, 0.0), axis=1, keepdims=True)
        sx2 = jnp.sum(jnp.where(onehot, x2, 0.0), axis=1, keepdims=True)
        sy2 = jnp.sum(jnp.where(onehot, y2, 0.0), axis=1, keepdims=True)
        xx1 = jnp.maximum(sx1, x1)
        yy1 = jnp.maximum(sy1, y1)
        xx2 = jnp.minimum(sx2, x2)
        yy2 = jnp.minimum(sy2, y2)
        inter = jnp.maximum(xx2 - xx1, 0.0) * jnp.maximum(yy2 - yy1, 0.0)
        a1 = jnp.maximum(sx2 - sx1, 0.0) * jnp.maximum(sy2 - sy1, 0.0)
        iou = inter / jnp.maximum(a1 + a2 - inter, 1e-8)
        kf = jnp.where(iou <= IOU_THR, af, 0.0)
        kf = jnp.where(onehot, 0.0, kf)
        af = jnp.where(valid, kf, af)
        oh = sstep == s
        vf = valid.astype(jnp.float32)
        os_ = os_ + jnp.where(oh, mx * vf, 0.0)
        ox1 = ox1 + jnp.where(oh, sx1 * vf, 0.0)
        oy1 = oy1 + jnp.where(oh, sy1 * vf, 0.0)
        ox2 = ox2 + jnp.where(oh, sx2 * vf, 0.0)
        oy2 = oy2 + jnp.where(oh, sy2 * vf, 0.0)
        return af, os_, ox1, oy1, ox2, oy2

    z = jnp.zeros((C, MAXDET), jnp.float32)
    active0 = jnp.where(scores > thr, 1.0, 0.0)
    _, os_, ox1, oy1, ox2, oy2 = lax.fori_loop(
        0, MAXDET, nms_step, (active0, z, z, z, z, z))

    flat = lax.broadcasted_iota(jnp.int32, (C, MAXDET), 0) * MAXDET + sstep
    out_lane = lax.broadcasted_iota(jnp.int32, (1, MAXDET), 1)

    def fin_step(k, carry):
        alive, fs, fx1, fy1, fx2, fy2, fc, nv = carry
        m = jnp.where(alive > 0.0, os_, -1.0)
        mx = jnp.max(m)
        fam = jnp.min(jnp.where(m == mx, flat, ROWS * MAXDET))
        oh = flat == fam
        bx1 = jnp.sum(jnp.where(oh, ox1, 0.0))
        by1 = jnp.sum(jnp.where(oh, oy1, 0.0))
        bx2 = jnp.sum(jnp.where(oh, ox2, 0.0))
        by2 = jnp.sum(jnp.where(oh, oy2, 0.0))
        valid = mx > thr
        vf = valid.astype(jnp.float32)
        cls = (fam // MAXDET).astype(jnp.float32)
        ohk = out_lane == k
        fs = fs + jnp.where(ohk, mx * vf, 0.0)
        fx1 = fx1 + jnp.where(ohk, jnp.clip(bx1, 0.0, 1.0) * vf, 0.0)
        fy1 = fy1 + jnp.where(ohk, jnp.clip(by1, 0.0, 1.0) * vf, 0.0)
        fx2 = fx2 + jnp.where(ohk, jnp.clip(bx2, 0.0, 1.0) * vf, 0.0)
        fy2 = fy2 + jnp.where(ohk, jnp.clip(by2, 0.0, 1.0) * vf, 0.0)
        fc = fc + jnp.where(ohk, cls, 0.0)
        alive = jnp.where(oh, 0.0, alive)
        return alive, fs, fx1, fy1, fx2, fy2, fc, nv + valid.astype(jnp.int32)

    zf = jnp.zeros((1, MAXDET), jnp.float32)
    alive0 = jnp.ones((C, MAXDET), jnp.float32)
    _, fs, fx1, fy1, fx2, fy2, fc, nv = lax.fori_loop(
        0, MAXDET, fin_step, (alive0, zf, zf, zf, zf, zf, zf, 0))
    fs_ref[...] = fs.reshape(1, 1, MAXDET)
    fx1_ref[...] = fx1.reshape(1, 1, MAXDET)
    fy1_ref[...] = fy1.reshape(1, 1, MAXDET)
    fx2_ref[...] = fx2.reshape(1, 1, MAXDET)
    fy2_ref[...] = fy2.reshape(1, 1, MAXDET)
    fc_ref[...] = fc.reshape(1, 1, MAXDET)
    fv_ref[...] = jnp.zeros((1, 1, 1), jnp.int32) + nv


def _nms_call(clog, cx1, cy1, cx2, cy2):
    spec_in = pl.BlockSpec((C, KPAD), lambda b: (b, 0))
    spec_o = pl.BlockSpec((1, 1, MAXDET), lambda b: (b, 0, 0))
    spec_v = pl.BlockSpec((1, 1, 1), lambda b: (b, 0, 0))
    f32 = jnp.float32
    out = pl.pallas_call(
        _nms_body,
        grid=(B,),
        in_specs=[spec_in] * 5,
        out_specs=[spec_o] * 6 + [spec_v],
        out_shape=[jax.ShapeDtypeStruct((B, 1, MAXDET), f32)] * 6
        + [jax.ShapeDtypeStruct((B, 1, 1), jnp.int32)],
    )(clog, cx1, cy1, cx2, cy2)
    return tuple(o.reshape(B, MAXDET) for o in out[:6]) + (out[6].reshape(B),)


@jax.jit
def kernel(predictions, anchors):
    predictions = predictions.astype(jnp.float32)
    logits_t = jnp.transpose(predictions[..., 4:], (0, 2, 1)).reshape(ROWS, N)
    loc = predictions[..., :4]
    cx, cy, w, h = (loc[..., i] for i in range(4))
    acx, acy, aw, ah = (anchors[:, i].reshape(1, N) for i in range(4))

    tb, bb = _thresh_call(logits_t)
    x1p, y1p, x2p, y2p = _decode_call(cx, cy, w, h, acx, acy, aw, ah)
    clog, cx1, cy1, cx2, cy2 = _sc_compact(logits_t, tb, bb, x1p, y1p, x2p, y2p)
    fs, fx1, fy1, fx2, fy2, fc, fv = _nms_call(clog, cx1, cy1, cx2, cy2)

    final_boxes = jnp.stack([fx1, fy1, fx2, fy2], axis=-1)
    return final_boxes, fc, fs, fv


# speedup vs baseline: 23.1909x; 1.3786x over previous
import functools

import jax
import jax.numpy as jnp
from jax import lax
from jax.experimental import pallas as pl
from jax.experimental.pallas import tpu as pltpu
from jax.experimental.pallas import tpu_sc as plsc

B = 8
N = 20000
C = 80
ROWS = B * C
K = 500
KPAD = 512
BUF = 544
MAXDET = 100
IOU_THR = 0.6
SCORE_THR = 0.1
NEG = -1e30


def _thresh_body(logit_ref, t_ref, bud_ref):
    x = logit_ref[...]
    bu = lax.bitcast_convert_type(x, jnp.uint32)
    topbit = jnp.uint32(0x80000000)
    ukey = jnp.where(bu >= topbit, ~bu, bu | topbit)

    def body(i, prefix):
        bit = jnp.uint32(31) - lax.convert_element_type(i, jnp.uint32)
        trial = prefix | lax.shift_left(jnp.uint32(1), bit)
        cnt = jnp.sum((ukey >= trial).astype(jnp.int32), axis=1, keepdims=True)
        return jnp.where(cnt >= K, trial, prefix)

    prefix = lax.fori_loop(0, 32, body, jnp.zeros((x.shape[0], 1), jnp.uint32))
    cnt_gt = jnp.sum((ukey > prefix).astype(jnp.int32), axis=1, keepdims=True)
    bits = jnp.where(prefix >= topbit, prefix ^ topbit, ~prefix)
    t = lax.bitcast_convert_type(bits, jnp.float32)
    t_ref[...] = jnp.broadcast_to(t, t_ref.shape)
    bud_ref[...] = jnp.broadcast_to(K - cnt_gt, bud_ref.shape)


def _thresh_call(logits_t):
    R = 64
    return pl.pallas_call(
        _thresh_body,
        grid=(ROWS // R,),
        in_specs=[pl.BlockSpec((R, N), lambda i: (i, 0))],
        out_specs=[
            pl.BlockSpec((R, 16), lambda i: (i, 0)),
            pl.BlockSpec((R, 16), lambda i: (i, 0)),
        ],
        out_shape=[
            jax.ShapeDtypeStruct((ROWS, 16), jnp.float32),
            jax.ShapeDtypeStruct((ROWS, 16), jnp.int32),
        ],
    )(logits_t)


def _decode_body(cx_ref, cy_ref, w_ref, h_ref, acx_ref, acy_ref, aw_ref, ah_ref,
                 x1_ref, y1_ref, x2_ref, y2_ref):
    v01 = jnp.float32(0.1)
    v2 = jnp.float32(0.2)
    aw = aw_ref[...]
    ah = ah_ref[...]
    xx = (cx_ref[...] * v01) * aw + acx_ref[...]
    yy = (cy_ref[...] * v01) * ah + acy_ref[...]
    ww = jnp.exp(w_ref[...] * v2) * aw
    hh = jnp.exp(h_ref[...] * v2) * ah
    x1_ref[...] = xx - ww / 2.0
    y1_ref[...] = yy - hh / 2.0
    x2_ref[...] = xx + ww / 2.0
    y2_ref[...] = yy + hh / 2.0


def _decode_call(cx, cy, w, h, acx, acy, aw, ah):
    spec_b = pl.BlockSpec((B, N), lambda: (0, 0))
    spec_a = pl.BlockSpec((1, N), lambda: (0, 0))
    return pl.pallas_call(
        _decode_body,
        in_specs=[spec_b] * 4 + [spec_a] * 4,
        out_specs=[spec_b] * 4,
        out_shape=[jax.ShapeDtypeStruct((B, N), jnp.float32)] * 4,
    )(cx, cy, w, h, acx, acy, aw, ah)


def _sc_body(logits_hbm, t_hbm, bud_hbm, x1_hbm, y1_hbm, x2_hbm, y2_hbm,
             clog_hbm, cx1_hbm, cy1_hbm, cx2_hbm, cy2_hbm,
             row_v, p1_v, p2_v, p3_v, p4_v, tb_v, bb_v,
             logit_b, idx_b, eq_b, o1_b, o2_b, o3_b, o4_b):
    wid = lax.axis_index("s") * 2 + lax.axis_index("c")
    batch = wid // 4
    row0 = batch * C + (wid % 4) * 20

    pltpu.sync_copy(x1_hbm.at[batch], p1_v)
    pltpu.sync_copy(y1_hbm.at[batch], p2_v)
    pltpu.sync_copy(x2_hbm.at[batch], p3_v)
    pltpu.sync_copy(y2_hbm.at[batch], p4_v)
    toff = row0 % 8
    tbase = pl.multiple_of(row0 - toff, 8)
    pltpu.sync_copy(t_hbm.at[pl.ds(tbase, 24), :], tb_v)
    pltpu.sync_copy(bud_hbm.at[pl.ds(tbase, 24), :], bb_v)

    iota = lax.iota(jnp.int32, 16)
    negv = jnp.full((16,), NEG, jnp.float32)
    zidx = jnp.zeros((16,), jnp.int32)

    def do_row(r, _):
        row = row0 + r
        pltpu.sync_copy(logits_hbm.at[row], row_v)
        t_vec = tb_v[toff + r, :]
        bud_vec = bb_v[toff + r, :]

        def init(j, _):
            logit_b[pl.ds(j * 16, 16)] = negv
            idx_b[pl.ds(j * 16, 16)] = zidx
            return 0

        lax.fori_loop(0, BUF // 16, init, 0)

        def step(i, carry):
            cur, ec = carry
            v = row_v[pl.ds(i * 16, 16)]
            m_gt = v > t_vec
            m_eq = v == t_vec
            any_sel = plsc.all_reduce_population_count(m_gt | m_eq)[0] > 0

            def taken(c):
                cur0, ec0 = c
                base = iota + i * 16
                plsc.store_compressed(idx_b.at[pl.ds(cur0, 16)], base, mask=m_gt)
                plsc.store_compressed(logit_b.at[pl.ds(cur0, 16)], v, mask=m_gt)
                ngt = plsc.all_reduce_population_count(m_gt)[0]

                def take_eq(e):
                    plsc.store_compressed(eq_b.at[pl.ds(e, 16)], base, mask=m_eq)
                    return e + plsc.all_reduce_population_count(m_eq)[0]

                ec1 = lax.cond(ec0 < KPAD, take_eq, lambda e: e, ec0)
                return (cur0 + ngt, ec1)

            return lax.cond(any_sel, taken, lambda c: c, (cur, ec))

        cur, _ec = lax.fori_loop(0, N // 16, step, (0, 0))

        budget = bud_vec[0]

        def fill(j, _):
            base = j * 16
            m = (iota + base) < bud_vec
            vi = eq_b[pl.ds(base, 16)]
            plsc.store_compressed(idx_b.at[pl.ds(cur + base, 16)], vi, mask=m)
            plsc.store_compressed(logit_b.at[pl.ds(cur + base, 16)], t_vec, mask=m)
            return 0

        lax.fori_loop(0, (budget + 15) // 16, fill, 0)

        def gather(j, _):
            s = pl.ds(j * 16, 16)
            vi = idx_b[s]
            o1_b[s] = plsc.load_gather(p1_v, [vi])
            o2_b[s] = plsc.load_gather(p2_v, [vi])
            o3_b[s] = plsc.load_gather(p3_v, [vi])
            o4_b[s] = plsc.load_gather(p4_v, [vi])
            return 0

        lax.fori_loop(0, KPAD // 16, gather, 0)

        pltpu.sync_copy(logit_b.at[pl.ds(0, KPAD)], clog_hbm.at[row])
        pltpu.sync_copy(o1_b, cx1_hbm.at[row])
        pltpu.sync_copy(o2_b, cy1_hbm.at[row])
        pltpu.sync_copy(o3_b, cx2_hbm.at[row])
        pltpu.sync_copy(o4_b, cy2_hbm.at[row])
        return 0

    lax.fori_loop(0, 20, do_row, 0)


def _sc_compact(logits_t, tb, bb, x1p, y1p, x2p, y2p):
    f32 = jnp.float32
    mesh = plsc.VectorSubcoreMesh(core_axis_name="c", subcore_axis_name="s")
    return pl.kernel(
        _sc_body,
        out_type=[jax.ShapeDtypeStruct((ROWS, KPAD), f32)] * 5,
        mesh=mesh,
        compiler_params=pltpu.CompilerParams(needs_layout_passes=False),
        scratch_types=[
            pltpu.VMEM((N,), f32),
            pltpu.VMEM((N,), f32),
            pltpu.VMEM((N,), f32),
            pltpu.VMEM((N,), f32),
            pltpu.VMEM((N,), f32),
            pltpu.VMEM((24, 16), f32),
            pltpu.VMEM((24, 16), jnp.int32),
            pltpu.VMEM((BUF,), f32),
            pltpu.VMEM((BUF,), jnp.int32),
            pltpu.VMEM((BUF,), jnp.int32),
            pltpu.VMEM((KPAD,), f32),
            pltpu.VMEM((KPAD,), f32),
            pltpu.VMEM((KPAD,), f32),
            pltpu.VMEM((KPAD,), f32),
        ],
    )(logits_t, tb, bb, x1p, y1p, x2p, y2p)


def _nms_body(clog_ref, cx1_ref, cy1_ref, cx2_ref, cy2_ref,
              fs_ref, fx1_ref, fy1_ref, fx2_ref, fy2_ref, fc_ref, fv_ref):
    logit = clog_ref[...]
    scores = 1.0 / (1.0 + jnp.exp(-logit))
    x1 = cx1_ref[...]
    y1 = cy1_ref[...]
    x2 = cx2_ref[...]
    y2 = cy2_ref[...]
    a2 = jnp.maximum(x2 - x1, 0.0) * jnp.maximum(y2 - y1, 0.0)
    lane = lax.broadcasted_iota(jnp.int32, (ROWS, KPAD), 1)
    thr = jnp.float32(SCORE_THR)

    sstep = lax.broadcasted_iota(jnp.int32, (ROWS, MAXDET), 1)

    def nms_step(s, carry):
        af, os_, ox1, oy1, ox2, oy2 = carry
        masked = jnp.where(af > 0.0, scores, -1.0)
        mx = jnp.max(masked, axis=1, keepdims=True)
        am = jnp.min(jnp.where(masked == mx, lane, N), axis=1, keepdims=True)
        onehot = lane == am
        valid = mx > thr
        sx1 = jnp.sum(jnp.where(onehot, x1, 0.0), axis=1, keepdims=True)
        sy1 = jnp.sum(jnp.where(onehot, y1, 0.0), axis=1, keepdims=True)
        sx2 = jnp.sum(jnp.where(onehot, x2, 0.0), axis=1, keepdims=True)
        sy2 = jnp.sum(jnp.where(onehot, y2, 0.0), axis=1, keepdims=True)
        xx1 = jnp.maximum(sx1, x1)
        yy1 = jnp.maximum(sy1, y1)
        xx2 = jnp.minimum(sx2, x2)
        yy2 = jnp.minimum(sy2, y2)
        inter = jnp.maximum(xx2 - xx1, 0.0) * jnp.maximum(yy2 - yy1, 0.0)
        a1 = jnp.maximum(sx2 - sx1, 0.0) * jnp.maximum(sy2 - sy1, 0.0)
        iou = inter / jnp.maximum(a1 + a2 - inter, 1e-8)
        kf = jnp.where(iou <= IOU_THR, af, 0.0)
        kf = jnp.where(onehot, 0.0, kf)
        af = jnp.where(valid, kf, af)
        oh = sstep == s
        vf = valid.astype(jnp.float32)
        os_ = os_ + jnp.where(oh, mx * vf, 0.0)
        ox1 = ox1 + jnp.where(oh, sx1 * vf, 0.0)
        oy1 = oy1 + jnp.where(oh, sy1 * vf, 0.0)
        ox2 = ox2 + jnp.where(oh, sx2 * vf, 0.0)
        oy2 = oy2 + jnp.where(oh, sy2 * vf, 0.0)
        return af, os_, ox1, oy1, ox2, oy2

    z = jnp.zeros((ROWS, MAXDET), jnp.float32)
    active0 = jnp.where(scores > thr, 1.0, 0.0)
    _, os_, ox1, oy1, ox2, oy2 = lax.fori_loop(
        0, MAXDET, nms_step, (active0, z, z, z, z, z))

    F = C * MAXDET
    r_os = os_.reshape(B, C, MAXDET)
    r_x1 = ox1.reshape(B, C, MAXDET)
    r_y1 = oy1.reshape(B, C, MAXDET)
    r_x2 = ox2.reshape(B, C, MAXDET)
    r_y2 = oy2.reshape(B, C, MAXDET)
    flat = (lax.broadcasted_iota(jnp.int32, (B, C, MAXDET), 1) * MAXDET
            + lax.broadcasted_iota(jnp.int32, (B, C, MAXDET), 2))
    out_lane = lax.broadcasted_iota(jnp.int32, (B, MAXDET), 1)

    def _red2(op, x):
        return op(op(x, axis=2, keepdims=True), axis=1, keepdims=True)

    def fin_step(k, carry):
        alive, fs, fx1, fy1, fx2, fy2, fc, nv = carry
        m = jnp.where(alive > 0.0, r_os, -1.0)
        mx = _red2(jnp.max, m)
        fam = _red2(jnp.min, jnp.where(m == mx, flat, F))
        oh = flat == fam
        bx1 = _red2(jnp.sum, jnp.where(oh, r_x1, 0.0))
        by1 = _red2(jnp.sum, jnp.where(oh, r_y1, 0.0))
        bx2 = _red2(jnp.sum, jnp.where(oh, r_x2, 0.0))
        by2 = _red2(jnp.sum, jnp.where(oh, r_y2, 0.0))
        valid = mx > thr
        vf = valid.astype(jnp.float32).reshape(B, 1)
        mx2 = mx.reshape(B, 1)
        cls = (fam // MAXDET).astype(jnp.float32).reshape(B, 1)
        ohk = out_lane == k
        fs = fs + jnp.where(ohk, mx2 * vf, 0.0)
        fx1 = fx1 + jnp.where(ohk, jnp.clip(bx1.reshape(B, 1), 0.0, 1.0) * vf, 0.0)
        fy1 = fy1 + jnp.where(ohk, jnp.clip(by1.reshape(B, 1), 0.0, 1.0) * vf, 0.0)
        fx2 = fx2 + jnp.where(ohk, jnp.clip(bx2.reshape(B, 1), 0.0, 1.0) * vf, 0.0)
        fy2 = fy2 + jnp.where(ohk, jnp.clip(by2.reshape(B, 1), 0.0, 1.0) * vf, 0.0)
        fc = fc + jnp.where(ohk, cls, 0.0)
        alive = jnp.where(oh, 0.0, alive)
        nv = nv + jnp.where(valid.reshape(B, 1), 1, 0)
        return alive, fs, fx1, fy1, fx2, fy2, fc, nv

    zf = jnp.zeros((B, MAXDET), jnp.float32)
    zi = jnp.zeros((B, 1), jnp.int32)
    alive0 = jnp.ones((B, C, MAXDET), jnp.float32)
    _, fs, fx1, fy1, fx2, fy2, fc, nv = lax.fori_loop(
        0, MAXDET, fin_step, (alive0, zf, zf, zf, zf, zf, zf, zi))
    fs_ref[...] = fs
    fx1_ref[...] = fx1
    fy1_ref[...] = fy1
    fx2_ref[...] = fx2
    fy2_ref[...] = fy2
    fc_ref[...] = fc
    fv_ref[...] = nv


def _nms_call(clog, cx1, cy1, cx2, cy2):
    spec_in = pl.BlockSpec((ROWS, KPAD), lambda: (0, 0))
    spec_o = pl.BlockSpec((B, MAXDET), lambda: (0, 0))
    spec_v = pl.BlockSpec((B, 1), lambda: (0, 0))
    f32 = jnp.float32
    out = pl.pallas_call(
        _nms_body,
        in_specs=[spec_in] * 5,
        out_specs=[spec_o] * 6 + [spec_v],
        out_shape=[jax.ShapeDtypeStruct((B, MAXDET), f32)] * 6
        + [jax.ShapeDtypeStruct((B, 1), jnp.int32)],
    )(clog, cx1, cy1, cx2, cy2)
    return out[:6] + (out[6].reshape(B),)


@jax.jit
def kernel(predictions, anchors):
    predictions = predictions.astype(jnp.float32)
    logits_t = jnp.transpose(predictions[..., 4:], (0, 2, 1)).reshape(ROWS, N)
    loc = predictions[..., :4]
    cx, cy, w, h = (loc[..., i] for i in range(4))
    acx, acy, aw, ah = (anchors[:, i].reshape(1, N) for i in range(4))

    tb, bb = _thresh_call(logits_t)
    x1p, y1p, x2p, y2p = _decode_call(cx, cy, w, h, acx, acy, aw, ah)
    clog, cx1, cy1, cx2, cy2 = _sc_compact(logits_t, tb, bb, x1p, y1p, x2p, y2p)
    fs, fx1, fy1, fx2, fy2, fc, fv = _nms_call(clog, cx1, cy1, cx2, cy2)

    final_boxes = jnp.stack([fx1, fy1, fx2, fy2], axis=-1)
    return final_boxes, fc, fs, fv


# speedup vs baseline: 24.6542x; 1.0631x over previous
import functools

import jax
import jax.numpy as jnp
from jax import lax
from jax.experimental import pallas as pl
from jax.experimental.pallas import tpu as pltpu
from jax.experimental.pallas import tpu_sc as plsc

B = 8
N = 20000
C = 80
ROWS = B * C
K = 500
KPAD = 512
BUF = 544
MAXDET = 100
IOU_THR = 0.6
SCORE_THR = 0.1
NEG = -1e30


def _thresh_body(logit_ref, t_ref, bud_ref):
    x = logit_ref[...]
    bu = lax.bitcast_convert_type(x, jnp.uint32)
    topbit = jnp.uint32(0x80000000)
    ukey = jnp.where(bu >= topbit, ~bu, bu | topbit)

    def body(i, prefix):
        bit = jnp.uint32(31) - lax.convert_element_type(i, jnp.uint32)
        trial = prefix | lax.shift_left(jnp.uint32(1), bit)
        cnt = jnp.sum((ukey >= trial).astype(jnp.int32), axis=1, keepdims=True)
        return jnp.where(cnt >= K, trial, prefix)

    prefix = lax.fori_loop(0, 32, body, jnp.zeros((x.shape[0], 1), jnp.uint32))
    cnt_gt = jnp.sum((ukey > prefix).astype(jnp.int32), axis=1, keepdims=True)
    bits = jnp.where(prefix >= topbit, prefix ^ topbit, ~prefix)
    t = lax.bitcast_convert_type(bits, jnp.float32)
    t_ref[...] = jnp.broadcast_to(t, t_ref.shape)
    bud_ref[...] = jnp.broadcast_to(K - cnt_gt, bud_ref.shape)


def _thresh_call(logits_t):
    R = 64
    return pl.pallas_call(
        _thresh_body,
        grid=(ROWS // R,),
        in_specs=[pl.BlockSpec((R, N), lambda i: (i, 0))],
        out_specs=[
            pl.BlockSpec((R, 16), lambda i: (i, 0)),
            pl.BlockSpec((R, 16), lambda i: (i, 0)),
        ],
        out_shape=[
            jax.ShapeDtypeStruct((ROWS, 16), jnp.float32),
            jax.ShapeDtypeStruct((ROWS, 16), jnp.int32),
        ],
    )(logits_t)


def _decode_body(cx_ref, cy_ref, w_ref, h_ref, acx_ref, acy_ref, aw_ref, ah_ref,
                 x1_ref, y1_ref, x2_ref, y2_ref):
    v01 = jnp.float32(0.1)
    v2 = jnp.float32(0.2)
    aw = aw_ref[...]
    ah = ah_ref[...]
    xx = (cx_ref[...] * v01) * aw + acx_ref[...]
    yy = (cy_ref[...] * v01) * ah + acy_ref[...]
    ww = jnp.exp(w_ref[...] * v2) * aw
    hh = jnp.exp(h_ref[...] * v2) * ah
    x1_ref[...] = xx - ww / 2.0
    y1_ref[...] = yy - hh / 2.0
    x2_ref[...] = xx + ww / 2.0
    y2_ref[...] = yy + hh / 2.0


def _decode_call(cx, cy, w, h, acx, acy, aw, ah):
    spec_b = pl.BlockSpec((B, N), lambda: (0, 0))
    spec_a = pl.BlockSpec((1, N), lambda: (0, 0))
    return pl.pallas_call(
        _decode_body,
        in_specs=[spec_b] * 4 + [spec_a] * 4,
        out_specs=[spec_b] * 4,
        out_shape=[jax.ShapeDtypeStruct((B, N), jnp.float32)] * 4,
    )(cx, cy, w, h, acx, acy, aw, ah)


def _sc_body(logits_hbm, t_hbm, bud_hbm, x1_hbm, y1_hbm, x2_hbm, y2_hbm,
             clog_hbm, cx1_hbm, cy1_hbm, cx2_hbm, cy2_hbm,
             row_v, p1_v, p2_v, p3_v, p4_v, tb_v, bb_v,
             logit_b, idx_b, eq_b, o1_b, o2_b, o3_b, o4_b):
    wid = lax.axis_index("s") * 2 + lax.axis_index("c")
    batch = wid // 4
    row0 = batch * C + (wid % 4) * 20

    pltpu.sync_copy(x1_hbm.at[batch], p1_v)
    pltpu.sync_copy(y1_hbm.at[batch], p2_v)
    pltpu.sync_copy(x2_hbm.at[batch], p3_v)
    pltpu.sync_copy(y2_hbm.at[batch], p4_v)
    toff = row0 % 8
    tbase = pl.multiple_of(row0 - toff, 8)
    pltpu.sync_copy(t_hbm.at[pl.ds(tbase, 24), :], tb_v)
    pltpu.sync_copy(bud_hbm.at[pl.ds(tbase, 24), :], bb_v)

    iota = lax.iota(jnp.int32, 16)
    negv = jnp.full((16,), NEG, jnp.float32)
    zidx = jnp.zeros((16,), jnp.int32)

    def do_row(r, _):
        row = row0 + r
        pltpu.sync_copy(logits_hbm.at[row], row_v)
        t_vec = tb_v[toff + r, :]
        bud_vec = bb_v[toff + r, :]

        def init(j, _):
            logit_b[pl.ds(j * 16, 16)] = negv
            idx_b[pl.ds(j * 16, 16)] = zidx
            return 0

        lax.fori_loop(0, BUF // 16, init, 0)

        ones16 = jnp.ones((16,), jnp.int32)

        def step(i, carry):
            cur, ec = carry
            v = row_v[pl.ds(i * 16, 16)]
            m_gt = v > t_vec
            m_eq = v == t_vec
            base = iota + i * 16
            pos = cur + plsc.cumsum(ones16, mask=m_gt) - 1
            plsc.store_scatter(idx_b, [pos], base, mask=m_gt)
            plsc.store_scatter(logit_b, [pos], v, mask=m_gt)
            cur = cur + plsc.all_reduce_population_count(m_gt)[0]
            neq = plsc.all_reduce_population_count(m_eq)[0]

            def take_eq(e):
                plsc.store_compressed(eq_b.at[pl.ds(e, 16)], base, mask=m_eq)
                return e + neq

            ec = lax.cond((neq > 0) & (ec < KPAD), take_eq, lambda e: e, ec)
            return (cur, ec)

        cur, _ec = lax.fori_loop(0, N // 16, step, (0, 0))

        budget = bud_vec[0]

        def fill(j, _):
            base = j * 16
            m = (iota + base) < bud_vec
            vi = eq_b[pl.ds(base, 16)]
            plsc.store_compressed(idx_b.at[pl.ds(cur + base, 16)], vi, mask=m)
            plsc.store_compressed(logit_b.at[pl.ds(cur + base, 16)], t_vec, mask=m)
            return 0

        lax.fori_loop(0, (budget + 15) // 16, fill, 0)

        def gather(j, _):
            s = pl.ds(j * 16, 16)
            vi = idx_b[s]
            o1_b[s] = plsc.load_gather(p1_v, [vi])
            o2_b[s] = plsc.load_gather(p2_v, [vi])
            o3_b[s] = plsc.load_gather(p3_v, [vi])
            o4_b[s] = plsc.load_gather(p4_v, [vi])
            return 0

        lax.fori_loop(0, KPAD // 16, gather, 0)

        pltpu.sync_copy(logit_b.at[pl.ds(0, KPAD)], clog_hbm.at[row])
        pltpu.sync_copy(o1_b, cx1_hbm.at[row])
        pltpu.sync_copy(o2_b, cy1_hbm.at[row])
        pltpu.sync_copy(o3_b, cx2_hbm.at[row])
        pltpu.sync_copy(o4_b, cy2_hbm.at[row])
        return 0

    lax.fori_loop(0, 20, do_row, 0)


def _sc_compact(logits_t, tb, bb, x1p, y1p, x2p, y2p):
    f32 = jnp.float32
    mesh = plsc.VectorSubcoreMesh(core_axis_name="c", subcore_axis_name="s")
    return pl.kernel(
        _sc_body,
        out_type=[jax.ShapeDtypeStruct((ROWS, KPAD), f32)] * 5,
        mesh=mesh,
        compiler_params=pltpu.CompilerParams(needs_layout_passes=False),
        scratch_types=[
            pltpu.VMEM((N,), f32),
            pltpu.VMEM((N,), f32),
            pltpu.VMEM((N,), f32),
            pltpu.VMEM((N,), f32),
            pltpu.VMEM((N,), f32),
            pltpu.VMEM((24, 16), f32),
            pltpu.VMEM((24, 16), jnp.int32),
            pltpu.VMEM((BUF,), f32),
            pltpu.VMEM((BUF,), jnp.int32),
            pltpu.VMEM((BUF,), jnp.int32),
            pltpu.VMEM((KPAD,), f32),
            pltpu.VMEM((KPAD,), f32),
            pltpu.VMEM((KPAD,), f32),
            pltpu.VMEM((KPAD,), f32),
        ],
    )(logits_t, tb, bb, x1p, y1p, x2p, y2p)


def _nms_body(clog_ref, cx1_ref, cy1_ref, cx2_ref, cy2_ref,
              fs_ref, fx1_ref, fy1_ref, fx2_ref, fy2_ref, fc_ref, fv_ref):
    logit = clog_ref[...]
    scores = 1.0 / (1.0 + jnp.exp(-logit))
    x1 = cx1_ref[...]
    y1 = cy1_ref[...]
    x2 = cx2_ref[...]
    y2 = cy2_ref[...]
    a2 = jnp.maximum(x2 - x1, 0.0) * jnp.maximum(y2 - y1, 0.0)
    lane = lax.broadcasted_iota(jnp.int32, (ROWS, KPAD), 1)
    thr = jnp.float32(SCORE_THR)

    sstep = lax.broadcasted_iota(jnp.int32, (ROWS, MAXDET), 1)

    def nms_step(s, carry):
        af, os_, ox1, oy1, ox2, oy2 = carry
        masked = jnp.where(af > 0.0, scores, -1.0)
        mx = jnp.max(masked, axis=1, keepdims=True)
        am = jnp.min(jnp.where(masked == mx, lane, N), axis=1, keepdims=True)
        onehot = lane == am
        valid = mx > thr
        sx1 = jnp.sum(jnp.where(onehot, x1, 0.0), axis=1, keepdims=True)
        sy1 = jnp.sum(jnp.where(onehot, y1, 0.0), axis=1, keepdims=True)
        sx2 = jnp.sum(jnp.where(onehot, x2, 0.0), axis=1, keepdims=True)
        sy2 = jnp.sum(jnp.where(onehot, y2, 0.0), axis=1, keepdims=True)
        xx1 = jnp.maximum(sx1, x1)
        yy1 = jnp.maximum(sy1, y1)
        xx2 = jnp.minimum(sx2, x2)
        yy2 = jnp.minimum(sy2, y2)
        inter = jnp.maximum(xx2 - xx1, 0.0) * jnp.maximum(yy2 - yy1, 0.0)
        a1 = jnp.maximum(sx2 - sx1, 0.0) * jnp.maximum(sy2 - sy1, 0.0)
        iou = inter / jnp.maximum(a1 + a2 - inter, 1e-8)
        kf = jnp.where(iou <= IOU_THR, af, 0.0)
        kf = jnp.where(onehot, 0.0, kf)
        af = jnp.where(valid, kf, af)
        oh = sstep == s
        vf = valid.astype(jnp.float32)
        os_ = os_ + jnp.where(oh, mx * vf, 0.0)
        ox1 = ox1 + jnp.where(oh, sx1 * vf, 0.0)
        oy1 = oy1 + jnp.where(oh, sy1 * vf, 0.0)
        ox2 = ox2 + jnp.where(oh, sx2 * vf, 0.0)
        oy2 = oy2 + jnp.where(oh, sy2 * vf, 0.0)
        return af, os_, ox1, oy1, ox2, oy2

    z = jnp.zeros((ROWS, MAXDET), jnp.float32)
    active0 = jnp.where(scores > thr, 1.0, 0.0)
    _, os_, ox1, oy1, ox2, oy2 = lax.fori_loop(
        0, MAXDET, nms_step, (active0, z, z, z, z, z))

    F = C * MAXDET
    r_os = os_.reshape(B, C, MAXDET)
    r_x1 = ox1.reshape(B, C, MAXDET)
    r_y1 = oy1.reshape(B, C, MAXDET)
    r_x2 = ox2.reshape(B, C, MAXDET)
    r_y2 = oy2.reshape(B, C, MAXDET)
    flat = (lax.broadcasted_iota(jnp.int32, (B, C, MAXDET), 1) * MAXDET
            + lax.broadcasted_iota(jnp.int32, (B, C, MAXDET), 2))
    out_lane = lax.broadcasted_iota(jnp.int32, (B, MAXDET), 1)

    def _red2(op, x):
        return op(op(x, axis=2, keepdims=True), axis=1, keepdims=True)

    def fin_step(k, carry):
        alive, fs, fx1, fy1, fx2, fy2, fc, nv = carry
        m = jnp.where(alive > 0.0, r_os, -1.0)
        mx = _red2(jnp.max, m)
        fam = _red2(jnp.min, jnp.where(m == mx, flat, F))
        oh = flat == fam
        bx1 = _red2(jnp.sum, jnp.where(oh, r_x1, 0.0))
        by1 = _red2(jnp.sum, jnp.where(oh, r_y1, 0.0))
        bx2 = _red2(jnp.sum, jnp.where(oh, r_x2, 0.0))
        by2 = _red2(jnp.sum, jnp.where(oh, r_y2, 0.0))
        valid = mx > thr
        vf = valid.astype(jnp.float32).reshape(B, 1)
        mx2 = mx.reshape(B, 1)
        cls = (fam // MAXDET).astype(jnp.float32).reshape(B, 1)
        ohk = out_lane == k
        fs = fs + jnp.where(ohk, mx2 * vf, 0.0)
        fx1 = fx1 + jnp.where(ohk, jnp.clip(bx1.reshape(B, 1), 0.0, 1.0) * vf, 0.0)
        fy1 = fy1 + jnp.where(ohk, jnp.clip(by1.reshape(B, 1), 0.0, 1.0) * vf, 0.0)
        fx2 = fx2 + jnp.where(ohk, jnp.clip(bx2.reshape(B, 1), 0.0, 1.0) * vf, 0.0)
        fy2 = fy2 + jnp.where(ohk, jnp.clip(by2.reshape(B, 1), 0.0, 1.0) * vf, 0.0)
        fc = fc + jnp.where(ohk, cls, 0.0)
        alive = jnp.where(oh, 0.0, alive)
        nv = nv + jnp.where(valid.reshape(B, 1), 1, 0)
        return alive, fs, fx1, fy1, fx2, fy2, fc, nv

    zf = jnp.zeros((B, MAXDET), jnp.float32)
    zi = jnp.zeros((B, 1), jnp.int32)
    alive0 = jnp.ones((B, C, MAXDET), jnp.float32)
    _, fs, fx1, fy1, fx2, fy2, fc, nv = lax.fori_loop(
        0, MAXDET, fin_step, (alive0, zf, zf, zf, zf, zf, zf, zi))
    fs_ref[...] = fs
    fx1_ref[...] = fx1
    fy1_ref[...] = fy1
    fx2_ref[...] = fx2
    fy2_ref[...] = fy2
    fc_ref[...] = fc
    fv_ref[...] = nv


def _nms_call(clog, cx1, cy1, cx2, cy2):
    spec_in = pl.BlockSpec((ROWS, KPAD), lambda: (0, 0))
    spec_o = pl.BlockSpec((B, MAXDET), lambda: (0, 0))
    spec_v = pl.BlockSpec((B, 1), lambda: (0, 0))
    f32 = jnp.float32
    out = pl.pallas_call(
        _nms_body,
        in_specs=[spec_in] * 5,
        out_specs=[spec_o] * 6 + [spec_v],
        out_shape=[jax.ShapeDtypeStruct((B, MAXDET), f32)] * 6
        + [jax.ShapeDtypeStruct((B, 1), jnp.int32)],
    )(clog, cx1, cy1, cx2, cy2)
    return out[:6] + (out[6].reshape(B),)


@jax.jit
def kernel(predictions, anchors):
    predictions = predictions.astype(jnp.float32)
    logits_t = jnp.transpose(predictions[..., 4:], (0, 2, 1)).reshape(ROWS, N)
    loc = predictions[..., :4]
    cx, cy, w, h = (loc[..., i] for i in range(4))
    acx, acy, aw, ah = (anchors[:, i].reshape(1, N) for i in range(4))

    tb, bb = _thresh_call(logits_t)
    x1p, y1p, x2p, y2p = _decode_call(cx, cy, w, h, acx, acy, aw, ah)
    clog, cx1, cy1, cx2, cy2 = _sc_compact(logits_t, tb, bb, x1p, y1p, x2p, y2p)
    fs, fx1, fy1, fx2, fy2, fc, fv = _nms_call(clog, cx1, cy1, cx2, cy2)

    final_boxes = jnp.stack([fx1, fy1, fx2, fy2], axis=-1)
    return final_boxes, fc, fs, fv


# speedup vs baseline: 26.5398x; 1.0765x over previous
import functools

import jax
import jax.numpy as jnp
from jax import lax
from jax.experimental import pallas as pl
from jax.experimental.pallas import tpu as pltpu
from jax.experimental.pallas import tpu_sc as plsc

B = 8
N = 20000
C = 80
ROWS = B * C
K = 500
KPAD = 512
BUF = 640
NPAD = 20096
MAXDET = 100
IOU_THR = 0.6
SCORE_THR = 0.1
NEG = -1e30


def _thresh_body(logit_ref, t_ref, bud_ref):
    x = logit_ref[...]
    bu = lax.bitcast_convert_type(x, jnp.uint32)
    topbit = jnp.uint32(0x80000000)
    ukey = jnp.where(bu >= topbit, ~bu, bu | topbit)

    def body(i, prefix):
        bit = jnp.uint32(31) - lax.convert_element_type(i, jnp.uint32)
        trial = prefix | lax.shift_left(jnp.uint32(1), bit)
        cnt = jnp.sum((ukey >= trial).astype(jnp.int32), axis=1, keepdims=True)
        return jnp.where(cnt >= K, trial, prefix)

    prefix = lax.fori_loop(0, 32, body, jnp.zeros((x.shape[0], 1), jnp.uint32))
    cnt_gt = jnp.sum((ukey > prefix).astype(jnp.int32), axis=1, keepdims=True)
    bits = jnp.where(prefix >= topbit, prefix ^ topbit, ~prefix)
    t = lax.bitcast_convert_type(bits, jnp.float32)
    t_ref[...] = jnp.broadcast_to(t, t_ref.shape)
    bud_ref[...] = jnp.broadcast_to(K - cnt_gt, bud_ref.shape)


def _thresh_call(logits_t):
    R = 64
    return pl.pallas_call(
        _thresh_body,
        grid=(ROWS // R,),
        in_specs=[pl.BlockSpec((R, N), lambda i: (i, 0))],
        out_specs=[
            pl.BlockSpec((R, 16), lambda i: (i, 0)),
            pl.BlockSpec((R, 16), lambda i: (i, 0)),
        ],
        out_shape=[
            jax.ShapeDtypeStruct((ROWS, 16), jnp.float32),
            jax.ShapeDtypeStruct((ROWS, 16), jnp.int32),
        ],
    )(logits_t)


def _decode_body(cx_ref, cy_ref, w_ref, h_ref, acx_ref, acy_ref, aw_ref, ah_ref,
                 x1_ref, y1_ref, x2_ref, y2_ref):
    v01 = jnp.float32(0.1)
    v2 = jnp.float32(0.2)
    aw = aw_ref[...]
    ah = ah_ref[...]
    xx = (cx_ref[...] * v01) * aw + acx_ref[...]
    yy = (cy_ref[...] * v01) * ah + acy_ref[...]
    ww = jnp.exp(w_ref[...] * v2) * aw
    hh = jnp.exp(h_ref[...] * v2) * ah
    x1_ref[...] = xx - ww / 2.0
    y1_ref[...] = yy - hh / 2.0
    x2_ref[...] = xx + ww / 2.0
    y2_ref[...] = yy + hh / 2.0


def _decode_call(cx, cy, w, h, acx, acy, aw, ah):
    spec_b = pl.BlockSpec((B, N), lambda: (0, 0))
    spec_a = pl.BlockSpec((1, N), lambda: (0, 0))
    return pl.pallas_call(
        _decode_body,
        in_specs=[spec_b] * 4 + [spec_a] * 4,
        out_specs=[spec_b] * 4,
        out_shape=[jax.ShapeDtypeStruct((B, N), jnp.float32)] * 4,
    )(cx, cy, w, h, acx, acy, aw, ah)


def _sc_body(logits_hbm, t_hbm, bud_hbm, x1_hbm, y1_hbm, x2_hbm, y2_hbm,
             clog_hbm, cx1_hbm, cy1_hbm, cx2_hbm, cy2_hbm,
             ch_v0, ch_v1, p1_v, p2_v, p3_v, p4_v, tb_v, bb_v,
             logit_b0, logit_b1, idx_b0, idx_b1, eq_b,
             o1_b0, o1_b1, o2_b0, o2_b1, o3_b0, o3_b1, o4_b0, o4_b1,
             sem_in, sem_out0, sem_out1):
    wid = lax.axis_index("s") * 2 + lax.axis_index("c")
    batch = wid // 4
    row0 = batch * C + (wid % 4) * 20

    pltpu.sync_copy(x1_hbm.at[batch], p1_v)
    pltpu.sync_copy(y1_hbm.at[batch], p2_v)
    pltpu.sync_copy(x2_hbm.at[batch], p3_v)
    pltpu.sync_copy(y2_hbm.at[batch], p4_v)
    toff = row0 % 8
    tbase = pl.multiple_of(row0 - toff, 8)
    pltpu.sync_copy(t_hbm.at[pl.ds(tbase, 24), :], tb_v)
    pltpu.sync_copy(bud_hbm.at[pl.ds(tbase, 24), :], bb_v)

    iota = lax.iota(jnp.int32, 16)
    negv = jnp.full((16,), NEG, jnp.float32)
    zidx = jnp.zeros((16,), jnp.int32)
    ones16 = jnp.ones((16,), jnp.int32)
    sems = (sem_out0, sem_out1)
    out_handles = {0: None, 1: None}

    chunks = (ch_v0, ch_v1)
    logits_b = (logit_b0, logit_b1)
    idxs_b = (idx_b0, idx_b1)
    outs_b = ((o1_b0, o2_b0, o3_b0, o4_b0), (o1_b1, o2_b1, o3_b1, o4_b1))
    NCH = 2
    CH = N // NCH

    h_in = pltpu.async_copy(
        logits_hbm.at[pl.ds(pl.multiple_of(row0 * N, 8), CH)], ch_v0, sem_in)
    for r in range(20):
        p = r % 2
        row = row0 + r
        lb = logits_b[p]
        ib = idxs_b[p]
        o1, o2, o3, o4 = outs_b[p]
        if out_handles[p] is not None:
            for h in out_handles[p]:
                h.wait()
        t_vec = tb_v[toff + r, :]
        bud_vec = bb_v[toff + r, :]

        def init(j, _):
            lb[pl.ds(j * 16, 16)] = negv
            ib[pl.ds(j * 16, 16)] = zidx
            return 0

        lax.fori_loop(0, BUF // 16, init, 0)

        carry = (0, 0)
        for c in range(NCH):
            q = r * NCH + c
            rv = chunks[q % 2]
            h_in.wait()
            if q < 20 * NCH - 1:
                nrow = row0 + (q + 1) // NCH
                noff = ((q + 1) % NCH) * CH
                h_in = pltpu.async_copy(
                    logits_hbm.at[pl.ds(pl.multiple_of(nrow * N + noff, 8), CH)],
                    chunks[(q + 1) % 2], sem_in)

            def step(i, carry):
                cur, ec = carry
                v = rv[pl.ds(i * 16, 16)]
                m_gt = v > t_vec
                m_eq = v == t_vec
                base = iota + (c * CH + i * 16)
                pos = cur + plsc.cumsum(ones16, mask=m_gt) - 1
                plsc.store_scatter(ib, [pos], base, mask=m_gt)
                plsc.store_scatter(lb, [pos], v, mask=m_gt)
                cur = cur + plsc.all_reduce_population_count(m_gt)[0]
                pos_e = jnp.minimum(
                    ec + plsc.cumsum(ones16, mask=m_eq) - 1, BUF - 1)
                plsc.store_scatter(eq_b, [pos_e], base, mask=m_eq)
                ec = ec + plsc.all_reduce_population_count(m_eq)[0]
                return (cur, ec)

            carry = lax.fori_loop(0, CH // 16, step, carry)
        cur, _ec = carry

        budget = bud_vec[0]

        def fill(j, _):
            base = j * 16
            m = (iota + base) < bud_vec
            vi = eq_b[pl.ds(base, 16)]
            plsc.store_compressed(ib.at[pl.ds(cur + base, 16)], vi, mask=m)
            plsc.store_compressed(lb.at[pl.ds(cur + base, 16)], t_vec, mask=m)
            return 0

        lax.fori_loop(0, (budget + 15) // 16, fill, 0)

        def gather(j, _):
            s = pl.ds(j * 16, 16)
            vi = ib[s]
            o1[s] = plsc.load_gather(p1_v, [vi])
            o2[s] = plsc.load_gather(p2_v, [vi])
            o3[s] = plsc.load_gather(p3_v, [vi])
            o4[s] = plsc.load_gather(p4_v, [vi])
            return 0

        lax.fori_loop(0, KPAD // 16, gather, 0)

        sem = sems[p]
        out_handles[p] = [
            pltpu.async_copy(lb.at[pl.ds(0, KPAD)], clog_hbm.at[row], sem),
            pltpu.async_copy(o1, cx1_hbm.at[row], sem),
            pltpu.async_copy(o2, cy1_hbm.at[row], sem),
            pltpu.async_copy(o3, cx2_hbm.at[row], sem),
            pltpu.async_copy(o4, cy2_hbm.at[row], sem),
        ]
    for p in (0, 1):
        for h in out_handles[p]:
            h.wait()


def _sc_compact(logits_t, tb, bb, x1p, y1p, x2p, y2p):
    f32 = jnp.float32
    i32 = jnp.int32
    mesh = plsc.VectorSubcoreMesh(core_axis_name="c", subcore_axis_name="s")
    return pl.kernel(
        _sc_body,
        out_type=[jax.ShapeDtypeStruct((ROWS, KPAD), f32)] * 5,
        mesh=mesh,
        compiler_params=pltpu.CompilerParams(needs_layout_passes=False),
        scratch_types=(
            [pltpu.VMEM((N // 2,), f32)] * 2
            + [pltpu.VMEM((N,), f32)] * 4
            + [pltpu.VMEM((24, 16), f32), pltpu.VMEM((24, 16), i32)]
            + [pltpu.VMEM((BUF,), f32)] * 2
            + [pltpu.VMEM((BUF,), i32)] * 2
            + [pltpu.VMEM((BUF,), i32)]
            + [pltpu.VMEM((KPAD,), f32)] * 8
            + [pltpu.SemaphoreType.DMA] * 3
        ),
    )(logits_t, tb, bb, x1p, y1p, x2p, y2p)


def _nms_body(clog_ref, cx1_ref, cy1_ref, cx2_ref, cy2_ref,
              fs_ref, fx1_ref, fy1_ref, fx2_ref, fy2_ref, fc_ref, fv_ref):
    logit = clog_ref[...]
    scores = 1.0 / (1.0 + jnp.exp(-logit))
    x1 = cx1_ref[...]
    y1 = cy1_ref[...]
    x2 = cx2_ref[...]
    y2 = cy2_ref[...]
    a2 = jnp.maximum(x2 - x1, 0.0) * jnp.maximum(y2 - y1, 0.0)
    lane = lax.broadcasted_iota(jnp.int32, (ROWS, KPAD), 1)
    thr = jnp.float32(SCORE_THR)

    sstep = lax.broadcasted_iota(jnp.int32, (ROWS, MAXDET), 1)

    def nms_step(s, carry):
        af, os_, ox1, oy1, ox2, oy2 = carry
        masked = jnp.where(af > 0.0, scores, -1.0)
        mx = jnp.max(masked, axis=1, keepdims=True)
        am = jnp.min(jnp.where(masked == mx, lane, N), axis=1, keepdims=True)
        onehot = lane == am
        valid = mx > thr
        sx1 = jnp.sum(jnp.where(onehot, x1, 0.0), axis=1, keepdims=True)
        sy1 = jnp.sum(jnp.where(onehot, y1, 0.0), axis=1, keepdims=True)
        sx2 = jnp.sum(jnp.where(onehot, x2, 0.0), axis=1, keepdims=True)
        sy2 = jnp.sum(jnp.where(onehot, y2, 0.0), axis=1, keepdims=True)
        xx1 = jnp.maximum(sx1, x1)
        yy1 = jnp.maximum(sy1, y1)
        xx2 = jnp.minimum(sx2, x2)
        yy2 = jnp.minimum(sy2, y2)
        inter = jnp.maximum(xx2 - xx1, 0.0) * jnp.maximum(yy2 - yy1, 0.0)
        a1 = jnp.maximum(sx2 - sx1, 0.0) * jnp.maximum(sy2 - sy1, 0.0)
        iou = inter / jnp.maximum(a1 + a2 - inter, 1e-8)
        kf = jnp.where(iou <= IOU_THR, af, 0.0)
        kf = jnp.where(onehot, 0.0, kf)
        af = jnp.where(valid, kf, af)
        oh = sstep == s
        vf = valid.astype(jnp.float32)
        os_ = os_ + jnp.where(oh, mx * vf, 0.0)
        ox1 = ox1 + jnp.where(oh, sx1 * vf, 0.0)
        oy1 = oy1 + jnp.where(oh, sy1 * vf, 0.0)
        ox2 = ox2 + jnp.where(oh, sx2 * vf, 0.0)
        oy2 = oy2 + jnp.where(oh, sy2 * vf, 0.0)
        return af, os_, ox1, oy1, ox2, oy2

    z = jnp.zeros((ROWS, MAXDET), jnp.float32)
    active0 = jnp.where(scores > thr, 1.0, 0.0)
    _, os_, ox1, oy1, ox2, oy2 = lax.fori_loop(
        0, MAXDET, nms_step, (active0, z, z, z, z, z))

    F = C * MAXDET
    r_os = os_.reshape(B, C, MAXDET)
    r_x1 = ox1.reshape(B, C, MAXDET)
    r_y1 = oy1.reshape(B, C, MAXDET)
    r_x2 = ox2.reshape(B, C, MAXDET)
    r_y2 = oy2.reshape(B, C, MAXDET)
    flat = (lax.broadcasted_iota(jnp.int32, (B, C, MAXDET), 1) * MAXDET
            + lax.broadcasted_iota(jnp.int32, (B, C, MAXDET), 2))
    out_lane = lax.broadcasted_iota(jnp.int32, (B, MAXDET), 1)

    def _red2(op, x):
        return op(op(x, axis=2, keepdims=True), axis=1, keepdims=True)

    def fin_step(k, carry):
        alive, fs, fx1, fy1, fx2, fy2, fc, nv = carry
        m = jnp.where(alive > 0.0, r_os, -1.0)
        mx = _red2(jnp.max, m)
        fam = _red2(jnp.min, jnp.where(m == mx, flat, F))
        oh = flat == fam
        bx1 = _red2(jnp.sum, jnp.where(oh, r_x1, 0.0))
        by1 = _red2(jnp.sum, jnp.where(oh, r_y1, 0.0))
        bx2 = _red2(jnp.sum, jnp.where(oh, r_x2, 0.0))
        by2 = _red2(jnp.sum, jnp.where(oh, r_y2, 0.0))
        valid = mx > thr
        vf = valid.astype(jnp.float32).reshape(B, 1)
        mx2 = mx.reshape(B, 1)
        cls = (fam // MAXDET).astype(jnp.float32).reshape(B, 1)
        ohk = out_lane == k
        fs = fs + jnp.where(ohk, mx2 * vf, 0.0)
        fx1 = fx1 + jnp.where(ohk, jnp.clip(bx1.reshape(B, 1), 0.0, 1.0) * vf, 0.0)
        fy1 = fy1 + jnp.where(ohk, jnp.clip(by1.reshape(B, 1), 0.0, 1.0) * vf, 0.0)
        fx2 = fx2 + jnp.where(ohk, jnp.clip(bx2.reshape(B, 1), 0.0, 1.0) * vf, 0.0)
        fy2 = fy2 + jnp.where(ohk, jnp.clip(by2.reshape(B, 1), 0.0, 1.0) * vf, 0.0)
        fc = fc + jnp.where(ohk, cls, 0.0)
        alive = jnp.where(oh, 0.0, alive)
        nv = nv + jnp.where(valid.reshape(B, 1), 1, 0)
        return alive, fs, fx1, fy1, fx2, fy2, fc, nv

    zf = jnp.zeros((B, MAXDET), jnp.float32)
    zi = jnp.zeros((B, 1), jnp.int32)
    alive0 = jnp.ones((B, C, MAXDET), jnp.float32)
    _, fs, fx1, fy1, fx2, fy2, fc, nv = lax.fori_loop(
        0, MAXDET, fin_step, (alive0, zf, zf, zf, zf, zf, zf, zi))
    fs_ref[...] = fs
    fx1_ref[...] = fx1
    fy1_ref[...] = fy1
    fx2_ref[...] = fx2
    fy2_ref[...] = fy2
    fc_ref[...] = fc
    fv_ref[...] = nv


def _nms_call(clog, cx1, cy1, cx2, cy2):
    spec_in = pl.BlockSpec((ROWS, KPAD), lambda: (0, 0))
    spec_o = pl.BlockSpec((B, MAXDET), lambda: (0, 0))
    spec_v = pl.BlockSpec((B, 1), lambda: (0, 0))
    f32 = jnp.float32
    out = pl.pallas_call(
        _nms_body,
        in_specs=[spec_in] * 5,
        out_specs=[spec_o] * 6 + [spec_v],
        out_shape=[jax.ShapeDtypeStruct((B, MAXDET), f32)] * 6
        + [jax.ShapeDtypeStruct((B, 1), jnp.int32)],
    )(clog, cx1, cy1, cx2, cy2)
    return out[:6] + (out[6].reshape(B),)


@jax.jit
def kernel(predictions, anchors):
    predictions = predictions.astype(jnp.float32)
    logits_t = jnp.transpose(predictions[..., 4:], (0, 2, 1)).reshape(ROWS, N)
    loc = predictions[..., :4]
    cx, cy, w, h = (loc[..., i] for i in range(4))
    acx, acy, aw, ah = (anchors[:, i].reshape(1, N) for i in range(4))

    tb, bb = _thresh_call(logits_t)
    x1p, y1p, x2p, y2p = _decode_call(cx, cy, w, h, acx, acy, aw, ah)
    clog, cx1, cy1, cx2, cy2 = _sc_compact(
        logits_t.reshape(-1), tb, bb, x1p, y1p, x2p, y2p)
    fs, fx1, fy1, fx2, fy2, fc, fv = _nms_call(clog, cx1, cy1, cx2, cy2)

    final_boxes = jnp.stack([fx1, fy1, fx2, fy2], axis=-1)
    return final_boxes, fc, fs, fv


# speedup vs baseline: 32.0632x; 1.2081x over previous
import functools

import jax
import jax.numpy as jnp
from jax import lax
from jax.experimental import pallas as pl
from jax.experimental.pallas import tpu as pltpu
from jax.experimental.pallas import tpu_sc as plsc

B = 8
N = 20000
C = 80
ROWS = B * C
K = 500
KPAD = 512
BUF = 640
NPAD = 20096
MAXDET = 100
IOU_THR = 0.6
SCORE_THR = 0.1
NEG = -1e30


def _thresh_body(logit_ref, t_ref, bud_ref):
    x = logit_ref[...]
    bu = lax.bitcast_convert_type(x, jnp.uint32)
    topbit = jnp.uint32(0x80000000)
    ukey = jnp.where(bu >= topbit, ~bu, bu | topbit)

    def body(i, prefix):
        bit = jnp.uint32(31) - lax.convert_element_type(i, jnp.uint32)
        trial = prefix | lax.shift_left(jnp.uint32(1), bit)
        cnt = jnp.sum((ukey >= trial).astype(jnp.int32), axis=1, keepdims=True)
        return jnp.where(cnt >= K, trial, prefix)

    prefix = lax.fori_loop(0, 32, body, jnp.zeros((x.shape[0], 1), jnp.uint32))
    cnt_gt = jnp.sum((ukey > prefix).astype(jnp.int32), axis=1, keepdims=True)
    bits = jnp.where(prefix >= topbit, prefix ^ topbit, ~prefix)
    t = lax.bitcast_convert_type(bits, jnp.float32)
    t_ref[...] = jnp.broadcast_to(t, t_ref.shape)
    bud_ref[...] = jnp.broadcast_to(K - cnt_gt, bud_ref.shape)


def _thresh_call(logits_t):
    R = 64
    nr = logits_t.shape[0]
    return pl.pallas_call(
        _thresh_body,
        grid=(nr // R,),
        in_specs=[pl.BlockSpec((R, N), lambda i: (i, 0))],
        out_specs=[
            pl.BlockSpec((R, 16), lambda i: (i, 0)),
            pl.BlockSpec((R, 16), lambda i: (i, 0)),
        ],
        out_shape=[
            jax.ShapeDtypeStruct((nr, 16), jnp.float32),
            jax.ShapeDtypeStruct((nr, 16), jnp.int32),
        ],
    )(logits_t)


def _decode_body(cx_ref, cy_ref, w_ref, h_ref, acx_ref, acy_ref, aw_ref, ah_ref,
                 x1_ref, y1_ref, x2_ref, y2_ref):
    v01 = jnp.float32(0.1)
    v2 = jnp.float32(0.2)
    aw = aw_ref[...]
    ah = ah_ref[...]
    xx = (cx_ref[...] * v01) * aw + acx_ref[...]
    yy = (cy_ref[...] * v01) * ah + acy_ref[...]
    ww = jnp.exp(w_ref[...] * v2) * aw
    hh = jnp.exp(h_ref[...] * v2) * ah
    x1_ref[...] = xx - ww / 2.0
    y1_ref[...] = yy - hh / 2.0
    x2_ref[...] = xx + ww / 2.0
    y2_ref[...] = yy + hh / 2.0


def _decode_call(cx, cy, w, h, acx, acy, aw, ah):
    spec_b = pl.BlockSpec((B, N), lambda: (0, 0))
    spec_a = pl.BlockSpec((1, N), lambda: (0, 0))
    return pl.pallas_call(
        _decode_body,
        in_specs=[spec_b] * 4 + [spec_a] * 4,
        out_specs=[spec_b] * 4,
        out_shape=[jax.ShapeDtypeStruct((B, N), jnp.float32)] * 4,
    )(cx, cy, w, h, acx, acy, aw, ah)


def _sc_body(half, nrows,
             logits_hbm, t_hbm, bud_hbm, x1_hbm, y1_hbm, x2_hbm, y2_hbm,
             clog_hbm, cx1_hbm, cy1_hbm, cx2_hbm, cy2_hbm,
             ch_v0, ch_v1, p1_v, p2_v, p3_v, p4_v, tb_v, bb_v,
             logit_b0, logit_b1, idx_b0, idx_b1, eq_b,
             o1_b0, o1_b1, o2_b0, o2_b1, o3_b0, o3_b1, o4_b0, o4_b1,
             sem_in, sem_out0, sem_out1):
    wid = lax.axis_index("s") * 2 + lax.axis_index("c")
    batch = half * 4 + wid // 8
    row0 = batch * C + (wid % 8) * nrows
    orow0 = row0 - half * 4 * C

    pltpu.sync_copy(x1_hbm.at[batch], p1_v)
    pltpu.sync_copy(y1_hbm.at[batch], p2_v)
    pltpu.sync_copy(x2_hbm.at[batch], p3_v)
    pltpu.sync_copy(y2_hbm.at[batch], p4_v)
    toff = orow0 % 8
    tbase = pl.multiple_of(orow0 - toff, 8)
    pltpu.sync_copy(t_hbm.at[pl.ds(tbase, 24), :], tb_v)
    pltpu.sync_copy(bud_hbm.at[pl.ds(tbase, 24), :], bb_v)

    iota = lax.iota(jnp.int32, 16)
    negv = jnp.full((16,), NEG, jnp.float32)
    zidx = jnp.zeros((16,), jnp.int32)
    ones16 = jnp.ones((16,), jnp.int32)
    sems = (sem_out0, sem_out1)
    out_handles = {0: None, 1: None}

    chunks = (ch_v0, ch_v1)
    logits_b = (logit_b0, logit_b1)
    idxs_b = (idx_b0, idx_b1)
    outs_b = ((o1_b0, o2_b0, o3_b0, o4_b0), (o1_b1, o2_b1, o3_b1, o4_b1))
    NCH = 2
    CH = N // NCH

    h_in = pltpu.async_copy(
        logits_hbm.at[pl.ds(pl.multiple_of(row0 * N, 8), CH)], ch_v0, sem_in)
    for r in range(nrows):
        p = r % 2
        row = row0 + r
        orow = orow0 + r
        lb = logits_b[p]
        ib = idxs_b[p]
        o1, o2, o3, o4 = outs_b[p]
        if out_handles[p] is not None:
            for h in out_handles[p]:
                h.wait()
        t_vec = tb_v[toff + r, :]
        bud_vec = bb_v[toff + r, :]

        def init(j, _):
            lb[pl.ds(j * 16, 16)] = negv
            ib[pl.ds(j * 16, 16)] = zidx
            return 0

        lax.fori_loop(0, BUF // 16, init, 0)

        carry = (0, 0)
        for c in range(NCH):
            q = r * NCH + c
            rv = chunks[q % 2]
            h_in.wait()
            if q < nrows * NCH - 1:
                nrow = row0 + (q + 1) // NCH
                noff = ((q + 1) % NCH) * CH
                h_in = pltpu.async_copy(
                    logits_hbm.at[pl.ds(pl.multiple_of(nrow * N + noff, 8), CH)],
                    chunks[(q + 1) % 2], sem_in)

            def step(i, carry):
                cur, ec = carry
                v = rv[pl.ds(i * 16, 16)]
                m_gt = v > t_vec
                m_eq = v == t_vec
                base = iota + (c * CH + i * 16)
                pos = cur + plsc.cumsum(ones16, mask=m_gt) - 1
                plsc.store_scatter(ib, [pos], base, mask=m_gt)
                plsc.store_scatter(lb, [pos], v, mask=m_gt)
                cur = cur + plsc.all_reduce_population_count(m_gt)[0]
                pos_e = jnp.minimum(
                    ec + plsc.cumsum(ones16, mask=m_eq) - 1, BUF - 1)
                plsc.store_scatter(eq_b, [pos_e], base, mask=m_eq)
                ec = ec + plsc.all_reduce_population_count(m_eq)[0]
                return (cur, ec)

            carry = lax.fori_loop(0, CH // 16, step, carry)
        cur, _ec = carry

        budget = bud_vec[0]

        def fill(j, _):
            base = j * 16
            m = (iota + base) < bud_vec
            vi = eq_b[pl.ds(base, 16)]
            plsc.store_compressed(ib.at[pl.ds(cur + base, 16)], vi, mask=m)
            plsc.store_compressed(lb.at[pl.ds(cur + base, 16)], t_vec, mask=m)
            return 0

        lax.fori_loop(0, (budget + 15) // 16, fill, 0)

        def gather(j, _):
            s = pl.ds(j * 16, 16)
            vi = ib[s]
            o1[s] = plsc.load_gather(p1_v, [vi])
            o2[s] = plsc.load_gather(p2_v, [vi])
            o3[s] = plsc.load_gather(p3_v, [vi])
            o4[s] = plsc.load_gather(p4_v, [vi])
            return 0

        lax.fori_loop(0, KPAD // 16, gather, 0)

        sem = sems[p]
        out_handles[p] = [
            pltpu.async_copy(lb.at[pl.ds(0, KPAD)], clog_hbm.at[orow], sem),
            pltpu.async_copy(o1, cx1_hbm.at[orow], sem),
            pltpu.async_copy(o2, cy1_hbm.at[orow], sem),
            pltpu.async_copy(o3, cx2_hbm.at[orow], sem),
            pltpu.async_copy(o4, cy2_hbm.at[orow], sem),
        ]
    for p in (0, 1):
        for h in out_handles[p]:
            h.wait()


def _sc_compact(half, logits_flat, tb, bb, x1p, y1p, x2p, y2p):
    f32 = jnp.float32
    i32 = jnp.int32
    mesh = plsc.VectorSubcoreMesh(core_axis_name="c", subcore_axis_name="s")
    return pl.kernel(
        functools.partial(_sc_body, half, 10),
        out_type=[jax.ShapeDtypeStruct((ROWS // 2, KPAD), f32)] * 5,
        mesh=mesh,
        compiler_params=pltpu.CompilerParams(needs_layout_passes=False),
        scratch_types=(
            [pltpu.VMEM((N // 2,), f32)] * 2
            + [pltpu.VMEM((N,), f32)] * 4
            + [pltpu.VMEM((24, 16), f32), pltpu.VMEM((24, 16), i32)]
            + [pltpu.VMEM((BUF,), f32)] * 2
            + [pltpu.VMEM((BUF,), i32)] * 2
            + [pltpu.VMEM((BUF,), i32)]
            + [pltpu.VMEM((KPAD,), f32)] * 8
            + [pltpu.SemaphoreType.DMA] * 3
        ),
    )(logits_flat, tb, bb, x1p, y1p, x2p, y2p)


def _nms_body(clog_ref, cx1_ref, cy1_ref, cx2_ref, cy2_ref,
              fs_ref, fx1_ref, fy1_ref, fx2_ref, fy2_ref, fc_ref, fv_ref):
    logit = clog_ref[...]
    R_ = logit.shape[0]
    B_ = R_ // C
    scores = 1.0 / (1.0 + jnp.exp(-logit))
    x1 = cx1_ref[...]
    y1 = cy1_ref[...]
    x2 = cx2_ref[...]
    y2 = cy2_ref[...]
    a2 = jnp.maximum(x2 - x1, 0.0) * jnp.maximum(y2 - y1, 0.0)
    lane = lax.broadcasted_iota(jnp.int32, (R_, KPAD), 1)
    thr = jnp.float32(SCORE_THR)

    sstep = lax.broadcasted_iota(jnp.int32, (R_, MAXDET), 1)

    def nms_step(s, carry):
        af, os_, ox1, oy1, ox2, oy2 = carry
        masked = jnp.where(af > 0.0, scores, -1.0)
        mx = jnp.max(masked, axis=1, keepdims=True)
        am = jnp.min(jnp.where(masked == mx, lane, N), axis=1, keepdims=True)
        onehot = lane == am
        valid = mx > thr
        sx1 = jnp.sum(jnp.where(onehot, x1, 0.0), axis=1, keepdims=True)
        sy1 = jnp.sum(jnp.where(onehot, y1, 0.0), axis=1, keepdims=True)
        sx2 = jnp.sum(jnp.where(onehot, x2, 0.0), axis=1, keepdims=True)
        sy2 = jnp.sum(jnp.where(onehot, y2, 0.0), axis=1, keepdims=True)
        xx1 = jnp.maximum(sx1, x1)
        yy1 = jnp.maximum(sy1, y1)
        xx2 = jnp.minimum(sx2, x2)
        yy2 = jnp.minimum(sy2, y2)
        inter = jnp.maximum(xx2 - xx1, 0.0) * jnp.maximum(yy2 - yy1, 0.0)
        a1 = jnp.maximum(sx2 - sx1, 0.0) * jnp.maximum(sy2 - sy1, 0.0)
        iou = inter / jnp.maximum(a1 + a2 - inter, 1e-8)
        kf = jnp.where(iou <= IOU_THR, af, 0.0)
        kf = jnp.where(onehot, 0.0, kf)
        af = jnp.where(valid, kf, af)
        oh = sstep == s
        vf = valid.astype(jnp.float32)
        os_ = os_ + jnp.where(oh, mx * vf, 0.0)
        ox1 = ox1 + jnp.where(oh, sx1 * vf, 0.0)
        oy1 = oy1 + jnp.where(oh, sy1 * vf, 0.0)
        ox2 = ox2 + jnp.where(oh, sx2 * vf, 0.0)
        oy2 = oy2 + jnp.where(oh, sy2 * vf, 0.0)
        return af, os_, ox1, oy1, ox2, oy2

    z = jnp.zeros((R_, MAXDET), jnp.float32)
    active0 = jnp.where(scores > thr, 1.0, 0.0)
    _, os_, ox1, oy1, ox2, oy2 = lax.fori_loop(
        0, MAXDET, nms_step, (active0, z, z, z, z, z))

    F = C * MAXDET
    r_os = os_.reshape(B_, C, MAXDET)
    r_x1 = ox1.reshape(B_, C, MAXDET)
    r_y1 = oy1.reshape(B_, C, MAXDET)
    r_x2 = ox2.reshape(B_, C, MAXDET)
    r_y2 = oy2.reshape(B_, C, MAXDET)
    flat = (lax.broadcasted_iota(jnp.int32, (B_, C, MAXDET), 1) * MAXDET
            + lax.broadcasted_iota(jnp.int32, (B_, C, MAXDET), 2))
    out_lane = lax.broadcasted_iota(jnp.int32, (B_, MAXDET), 1)

    def _red2(op, x):
        return op(op(x, axis=2, keepdims=True), axis=1, keepdims=True)

    def fin_step(k, carry):
        alive, fs, fx1, fy1, fx2, fy2, fc, nv = carry
        m = jnp.where(alive > 0.0, r_os, -1.0)
        mx = _red2(jnp.max, m)
        fam = _red2(jnp.min, jnp.where(m == mx, flat, F))
        oh = flat == fam
        bx1 = _red2(jnp.sum, jnp.where(oh, r_x1, 0.0))
        by1 = _red2(jnp.sum, jnp.where(oh, r_y1, 0.0))
        bx2 = _red2(jnp.sum, jnp.where(oh, r_x2, 0.0))
        by2 = _red2(jnp.sum, jnp.where(oh, r_y2, 0.0))
        valid = mx > thr
        vf = valid.astype(jnp.float32).reshape(B_, 1)
        mx2 = mx.reshape(B_, 1)
        cls = (fam // MAXDET).astype(jnp.float32).reshape(B_, 1)
        ohk = out_lane == k
        fs = fs + jnp.where(ohk, mx2 * vf, 0.0)
        fx1 = fx1 + jnp.where(ohk, jnp.clip(bx1.reshape(B_, 1), 0.0, 1.0) * vf, 0.0)
        fy1 = fy1 + jnp.where(ohk, jnp.clip(by1.reshape(B_, 1), 0.0, 1.0) * vf, 0.0)
        fx2 = fx2 + jnp.where(ohk, jnp.clip(bx2.reshape(B_, 1), 0.0, 1.0) * vf, 0.0)
        fy2 = fy2 + jnp.where(ohk, jnp.clip(by2.reshape(B_, 1), 0.0, 1.0) * vf, 0.0)
        fc = fc + jnp.where(ohk, cls, 0.0)
        alive = jnp.where(oh, 0.0, alive)
        nv = nv + jnp.where(valid.reshape(B_, 1), 1, 0)
        return alive, fs, fx1, fy1, fx2, fy2, fc, nv

    zf = jnp.zeros((B_, MAXDET), jnp.float32)
    zi = jnp.zeros((B_, 1), jnp.int32)
    alive0 = jnp.ones((B_, C, MAXDET), jnp.float32)
    _, fs, fx1, fy1, fx2, fy2, fc, nv = lax.fori_loop(
        0, MAXDET, fin_step, (alive0, zf, zf, zf, zf, zf, zf, zi))
    fs_ref[...] = fs
    fx1_ref[...] = fx1
    fy1_ref[...] = fy1
    fx2_ref[...] = fx2
    fy2_ref[...] = fy2
    fc_ref[...] = fc
    fv_ref[...] = nv


def _nms_call(clog, cx1, cy1, cx2, cy2):
    nr = clog.shape[0]
    nb = nr // C
    spec_in = pl.BlockSpec((nr, KPAD), lambda: (0, 0))
    spec_o = pl.BlockSpec((nb, MAXDET), lambda: (0, 0))
    spec_v = pl.BlockSpec((nb, 1), lambda: (0, 0))
    f32 = jnp.float32
    out = pl.pallas_call(
        _nms_body,
        in_specs=[spec_in] * 5,
        out_specs=[spec_o] * 6 + [spec_v],
        out_shape=[jax.ShapeDtypeStruct((nb, MAXDET), f32)] * 6
        + [jax.ShapeDtypeStruct((nb, 1), jnp.int32)],
    )(clog, cx1, cy1, cx2, cy2)
    return out[:6] + (out[6].reshape(nb),)


@jax.jit
def kernel(predictions, anchors):
    predictions = predictions.astype(jnp.float32)
    logits_t = jnp.transpose(predictions[..., 4:], (0, 2, 1)).reshape(ROWS, N)
    loc = predictions[..., :4]
    cx, cy, w, h = (loc[..., i] for i in range(4))
    acx, acy, aw, ah = (anchors[:, i].reshape(1, N) for i in range(4))

    x1p, y1p, x2p, y2p = _decode_call(cx, cy, w, h, acx, acy, aw, ah)
    logits_flat = logits_t.reshape(-1)
    H = ROWS // 2
    halves = []
    for h_ in (0, 1):
        tb, bb = _thresh_call(logits_t[h_ * H:(h_ + 1) * H])
        cand = _sc_compact(h_, logits_flat, tb, bb, x1p, y1p, x2p, y2p)
        halves.append(_nms_call(*cand))

    outs = [jnp.concatenate([a, b], axis=0) for a, b in zip(halves[0], halves[1])]
    fs, fx1, fy1, fx2, fy2, fc, fv = outs
    final_boxes = jnp.stack([fx1, fy1, fx2, fy2], axis=-1)
    return final_boxes, fc, fs, fv
